# baseline (speedup 1.0000x reference)
import functools

import jax
import jax.numpy as jnp
from jax import lax
from jax.experimental import pallas as pl
from jax.experimental.pallas import tpu as pltpu
from jax.experimental.pallas import tpu_sc as plsc

K = 16
P = 1024
B = 8
EPS = 1e-5

NUM_SC_CORES = 2
NUM_SUBCORES = 16
NUM_TILES = NUM_SC_CORES * NUM_SUBCORES
LANES = 16


def _lrelu(x):
    return jnp.where(x >= 0, x, 0.2 * x)


_ROWS = 512


def _knn_body(pr_ref, pc_ref, out_ref):
    rows = pr_ref[0]
    cols = pc_ref[0]
    d = ((rows[:, 0:1] - cols[0:1, :]) ** 2
         + (rows[:, 1:2] - cols[1:2, :]) ** 2
         + (rows[:, 2:3] - cols[2:3, :]) ** 2)
    d = jnp.maximum(d, jnp.float32(2.0**-126))
    col_iota = lax.broadcasted_iota(jnp.int32, (_ROWS, P), 1)
    keys = lax.bitcast_convert_type(
        (lax.bitcast_convert_type(d, jnp.int32) & jnp.int32(~1023)) | col_iota,
        jnp.float32)
    inf = jnp.float32(jnp.inf)
    for k in range(K):
        m = jnp.min(keys, axis=1)
        out_ref[0, k, :] = (lax.bitcast_convert_type(m, jnp.int32)
                            & jnp.int32(1023))
        keys = jnp.where(keys == m[:, None], inf, keys)


def _knn(pos_r, pos_c):
    return pl.pallas_call(
        _knn_body,
        grid=(B, P // _ROWS),
        in_specs=[
            pl.BlockSpec((1, _ROWS, 3), lambda b, r: (b, r, 0)),
            pl.BlockSpec((1, 3, P), lambda b, r: (b, 0, 0)),
        ],
        out_specs=pl.BlockSpec((1, K, _ROWS), lambda b, r: (b, 0, r)),
        out_shape=jax.ShapeDtypeStruct((B, K, P), jnp.int32),
    )(pos_r, pos_c)


def _layer_mm(xa, xm, wd, wj, beta, cin, cout, fuse_in):
    def body(xa_ref, xm_ref, wd_ref, wj_ref, bt_ref, a_ref, bm_ref):
        if fuse_in:
            x = _lrelu(xa_ref[0] + xm_ref[0])
        else:
            x = xa_ref[0]
        if cin <= 4:
            a = jnp.zeros((cout, P), jnp.float32)
            bm = jnp.zeros((cout, P), jnp.float32)
            for c in range(cin):
                a = a + wd_ref[:, c:c + 1] * x[c:c + 1, :]
                bm = bm + wj_ref[:, c:c + 1] * x[c:c + 1, :]
        else:
            a = jnp.dot(wd_ref[...], x, preferred_element_type=jnp.float32)
            bm = jnp.dot(wj_ref[...], x, preferred_element_type=jnp.float32)
        a_ref[0] = a + bt_ref[...]
        bm_ref[0] = bm

    return pl.pallas_call(
        body,
        grid=(B,),
        in_specs=[
            pl.BlockSpec((1, cin, P), lambda b: (b, 0, 0)),
            pl.BlockSpec((1, cin, P), lambda b: (b, 0, 0)),
            pl.BlockSpec((cout, cin), lambda b: (0, 0)),
            pl.BlockSpec((cout, cin), lambda b: (0, 0)),
            pl.BlockSpec((cout, 1), lambda b: (0, 0)),
        ],
        out_specs=[
            pl.BlockSpec((1, cout, P), lambda b: (b, 0, 0)),
            pl.BlockSpec((1, cout, P), lambda b: (b, 0, 0)),
        ],
        out_shape=[
            jax.ShapeDtypeStruct((B, cout, P), jnp.float32),
            jax.ShapeDtypeStruct((B, cout, P), jnp.float32),
        ],
    )(xa, xm, wd, wj, beta)


_SC_CH = {64: 16, 128: 32, 256: 32}


def _sc_gather_max(bm_t, idx_t, cout):
    ch_sz = _SC_CH[cout]
    nch = cout // ch_sz
    ntasks = B * nch
    tasks_per_tile = ntasks // NUM_TILES
    mesh = plsc.VectorSubcoreMesh(core_axis_name="c", subcore_axis_name="s")

    @functools.partial(
        pl.kernel,
        mesh=mesh,
        out_type=jax.ShapeDtypeStruct((B, cout, P), jnp.float32),
        scratch_types=[
            pltpu.VMEM((ch_sz, P), jnp.float32),
            pltpu.VMEM((K, P), jnp.int32),
            pltpu.VMEM((ch_sz, P), jnp.float32),
        ],
        compiler_params=pltpu.CompilerParams(needs_layout_passes=False),
    )
    def k(bm_hbm, idx_hbm, out_hbm, bm_v, idx_v, out_v):
        wid = lax.axis_index("s") * NUM_SC_CORES + lax.axis_index("c")
        lane = lax.iota(jnp.int32, LANES)
        b = (wid * tasks_per_tile) // nch
        pltpu.sync_copy(idx_hbm.at[b], idx_v)
        for i in range(tasks_per_tile):
            t = wid * tasks_per_tile + i
            c0 = (t % nch) * ch_sz
            pltpu.sync_copy(bm_hbm.at[b, pl.ds(c0, ch_sz)], bm_v)

            def pg_body(pg, _):
                pvec = pg * LANES + lane
                jv = [plsc.load_gather(
                          idx_v, [jnp.full((LANES,), kk, jnp.int32), pvec])
                      for kk in range(K)]

                def c_body(c4, _):
                    for dc in range(4):
                        c = c4 * 4 + dc
                        cvec = jnp.broadcast_to(c, (LANES,)).astype(jnp.int32)
                        m = jnp.full((LANES,), -jnp.inf, jnp.float32)
                        for kk in range(K):
                            v = plsc.load_gather(bm_v, [cvec, jv[kk]])
                            m = jnp.maximum(m, v)
                        plsc.store_scatter(out_v, [cvec, pvec], m)
                    return 0

                lax.fori_loop(0, ch_sz // 4, c_body, 0)
                return 0

            lax.fori_loop(0, P // LANES, pg_body, 0)
            pltpu.sync_copy(out_v, out_hbm.at[b, pl.ds(c0, ch_sz)])

    return k(bm_t, idx_t)


def _conv5_head(ams, w5, b5, w6, b6, w7, b7, w8, b8):
    def body(a1, m1, a2, m2, a3, m3, a4, m4, w5_ref, b5_ref,
             w6_ref, b6_ref, w7_ref, b7_ref, w8_ref, b8_ref,
             out_ref, pooled_v):
        bidx = pl.program_id(0)
        xcat = jnp.concatenate(
            [_lrelu(a1[0] + m1[0]), _lrelu(a2[0] + m2[0]),
             _lrelu(a3[0] + m3[0]), _lrelu(a4[0] + m4[0])], axis=0)
        h = jnp.dot(w5_ref[...].astype(jnp.bfloat16),
                    xcat.astype(jnp.bfloat16),
                    preferred_element_type=jnp.float32)
        h = _lrelu(h + b5_ref[...])
        mx = jnp.max(h, axis=1)
        av = jnp.sum(h, axis=1) * jnp.float32(1.0 / P)
        pooled_v[pl.ds(bidx, 1), :] = jnp.concatenate([mx, av], axis=0)[None, :]

        @pl.when(bidx == B - 1)
        def _():
            dn = (((1,), (1,)), ((), ()))
            z = lax.dot_general(pooled_v[...], w6_ref[...], dn,
                                preferred_element_type=jnp.float32)
            z = _lrelu(z + b6_ref[...])
            z = lax.dot_general(z, w7_ref[...], dn,
                                preferred_element_type=jnp.float32)
            z = _lrelu(z + b7_ref[...])
            z = lax.dot_general(z, w8_ref[...], dn,
                                preferred_element_type=jnp.float32)
            out_ref[...] = z + b8_ref[...]

    specs = []
    for c in (64, 64, 128, 256):
        specs.append(pl.BlockSpec((1, c, P), lambda b: (b, 0, 0)))
        specs.append(pl.BlockSpec((1, c, P), lambda b: (b, 0, 0)))
    for shp in ((1024, 512), (1024, 1), (512, 2048), (1, 512),
                (256, 512), (1, 256), (40, 256), (1, 40)):
        specs.append(pl.BlockSpec(shp, lambda b: (0, 0)))
    return pl.pallas_call(
        body,
        grid=(B,),
        in_specs=specs,
        out_specs=pl.BlockSpec((B, 40), lambda b: (0, 0)),
        out_shape=jax.ShapeDtypeStruct((B, 40), jnp.float32),
        scratch_shapes=[pltpu.VMEM((B, 2048), jnp.float32)],
    )(*ams, w5, b5, w6, b6, w7, b7, w8, b8)


def kernel(pos, batch, W1, g1, b1, W2, g2, b2, W3, g3, b3, W4, g4, b4,
           W5, g5, b5, W6, g6, b6, W7, bias7, g7, b7, W8, bias8):
    del batch
    pos_r = pos.reshape(B, P, 3)
    pos_c = jnp.transpose(pos_r, (0, 2, 1))

    idx_t = _knn(pos_r, pos_c)

    inv = 1.0 / jnp.sqrt(jnp.float32(1.0 + EPS))

    def prep(W, g, bb, cin):
        ws = W * (g * inv)[:, None]
        wi, wj = ws[:, :cin], ws[:, cin:]
        return wi - wj, wj, bb[:, None]

    ams = []
    xa, xm = pos_c, pos_c
    fuse = False
    for (W, g, bb, cin, cout) in ((W1, g1, b1, 3, 64),
                                  (W2, g2, b2, 64, 64),
                                  (W3, g3, b3, 64, 128),
                                  (W4, g4, b4, 128, 256)):
        wd, wj, bt = prep(W, g, bb, cin)
        a_t, bm_t = _layer_mm(xa, xm, wd, wj, bt, cin, cout, fuse)
        m_t = _sc_gather_max(bm_t, idx_t, cout)
        ams.extend([a_t, m_t])
        xa, xm, fuse = a_t, m_t, True

    w5s = W5 * (g5 * inv)[:, None]
    w6s = W6 * (g6 * inv)[:, None]
    w7s = W7 * (g7 * inv)[:, None]
    b7s = (bias7 * g7 * inv + b7)
    return _conv5_head(ams, w5s, b5[:, None], w6s, b6[None, :],
                       w7s, b7s[None, :], W8, bias8[None, :])


# speedup vs baseline: 1.8293x; 1.8293x over previous
import functools

import jax
import jax.numpy as jnp
from jax import lax
from jax.experimental import pallas as pl
from jax.experimental.pallas import tpu as pltpu
from jax.experimental.pallas import tpu_sc as plsc

K = 16
P = 1024
B = 8
EPS = 1e-5

NUM_SC_CORES = 2
NUM_SUBCORES = 16
NUM_TILES = NUM_SC_CORES * NUM_SUBCORES
LANES = 16


def _lrelu(x):
    return jnp.where(x >= 0, x, 0.2 * x)


_ROWS = 512


def _knn_body(pr_ref, pc_ref, out_ref):
    rows = pr_ref[0]
    cols = pc_ref[0]
    d = ((rows[:, 0:1] - cols[0:1, :]) ** 2
         + (rows[:, 1:2] - cols[1:2, :]) ** 2
         + (rows[:, 2:3] - cols[2:3, :]) ** 2)
    d = jnp.maximum(d, jnp.float32(2.0**-126))
    col_iota = lax.broadcasted_iota(jnp.int32, (_ROWS, P), 1)
    keys = lax.bitcast_convert_type(
        (lax.bitcast_convert_type(d, jnp.int32) & jnp.int32(~1023)) | col_iota,
        jnp.float32)
    inf = jnp.float32(jnp.inf)
    for k in range(K):
        m = jnp.min(keys, axis=1)
        out_ref[0, k, :] = (lax.bitcast_convert_type(m, jnp.int32)
                            & jnp.int32(1023))
        keys = jnp.where(keys == m[:, None], inf, keys)


def _knn(pos_r, pos_c):
    return pl.pallas_call(
        _knn_body,
        grid=(B, P // _ROWS),
        in_specs=[
            pl.BlockSpec((1, _ROWS, 3), lambda b, r: (b, r, 0)),
            pl.BlockSpec((1, 3, P), lambda b, r: (b, 0, 0)),
        ],
        out_specs=pl.BlockSpec((1, K, _ROWS), lambda b, r: (b, 0, r)),
        out_shape=jax.ShapeDtypeStruct((B, K, P), jnp.int32),
    )(pos_r, pos_c)


def _layer_mm(xa, xm, wd, wj, beta, cin, cout, fuse_in):
    def body(xa_ref, xm_ref, wd_ref, wj_ref, bt_ref, a_ref, bm_ref):
        if fuse_in:
            x = _lrelu(xa_ref[0] + xm_ref[0])
        else:
            x = xa_ref[0]
        if cin <= 4:
            a = jnp.zeros((cout, P), jnp.float32)
            bm = jnp.zeros((cout, P), jnp.float32)
            for c in range(cin):
                a = a + wd_ref[:, c:c + 1] * x[c:c + 1, :]
                bm = bm + wj_ref[:, c:c + 1] * x[c:c + 1, :]
        else:
            a = jnp.dot(wd_ref[...], x, preferred_element_type=jnp.float32)
            bm = jnp.dot(wj_ref[...], x, preferred_element_type=jnp.float32)
        a_ref[0] = a + bt_ref[...]
        bm_ref[0] = bm

    return pl.pallas_call(
        body,
        grid=(B,),
        in_specs=[
            pl.BlockSpec((1, cin, P), lambda b: (b, 0, 0)),
            pl.BlockSpec((1, cin, P), lambda b: (b, 0, 0)),
            pl.BlockSpec((cout, cin), lambda b: (0, 0)),
            pl.BlockSpec((cout, cin), lambda b: (0, 0)),
            pl.BlockSpec((cout, 1), lambda b: (0, 0)),
        ],
        out_specs=[
            pl.BlockSpec((1, cout, P), lambda b: (b, 0, 0)),
            pl.BlockSpec((1, cout, P), lambda b: (b, 0, 0)),
        ],
        out_shape=[
            jax.ShapeDtypeStruct((B, cout, P), jnp.float32),
            jax.ShapeDtypeStruct((B, cout, P), jnp.float32),
        ],
    )(xa, xm, wd, wj, beta)


_SC_CH = {64: 16, 128: 32, 256: 32}


def _sc_gather_max(bm_t, idx_t, cout):
    ch_sz = _SC_CH[cout]
    nch = cout // ch_sz
    ntasks = B * nch
    tasks_per_tile = ntasks // NUM_TILES
    mesh = plsc.VectorSubcoreMesh(core_axis_name="c", subcore_axis_name="s")

    @functools.partial(
        pl.kernel,
        mesh=mesh,
        out_type=jax.ShapeDtypeStruct((B, cout, P), jnp.float32),
        scratch_types=[
            pltpu.VMEM((ch_sz, P), jnp.float32),
            pltpu.VMEM((K, P), jnp.int32),
            pltpu.VMEM((ch_sz, P), jnp.float32),
        ],
        compiler_params=pltpu.CompilerParams(needs_layout_passes=False),
    )
    def k(bm_hbm, idx_hbm, out_hbm, bm_v, idx_v, out_v):
        wid = lax.axis_index("s") * NUM_SC_CORES + lax.axis_index("c")
        lane = lax.iota(jnp.int32, LANES)
        b = (wid * tasks_per_tile) // nch
        pltpu.sync_copy(idx_hbm.at[b], idx_v)
        for i in range(tasks_per_tile):
            t = wid * tasks_per_tile + i
            c0 = (t % nch) * ch_sz
            pltpu.sync_copy(bm_hbm.at[b, pl.ds(c0, ch_sz)], bm_v)

            def pg_body(pg, _):
                pvec = pg * LANES + lane
                jv = [plsc.load_gather(
                          idx_v, [jnp.full((LANES,), kk, jnp.int32), pvec])
                      for kk in range(K)]

                def c_body(c, _):
                    cvec = jnp.broadcast_to(c, (LANES,)).astype(jnp.int32)
                    m = jnp.full((LANES,), -jnp.inf, jnp.float32)
                    for kk in range(K):
                        v = plsc.load_gather(bm_v, [cvec, jv[kk]])
                        m = jnp.maximum(m, v)
                    plsc.store_scatter(out_v, [cvec, pvec], m)
                    return 0

                lax.fori_loop(0, ch_sz, c_body, 0)
                return 0

            lax.fori_loop(0, P // LANES, pg_body, 0)
            pltpu.sync_copy(out_v, out_hbm.at[b, pl.ds(c0, ch_sz)])

    return k(bm_t, idx_t)


def _conv5_head(ams, w5, b5, w6, b6, w7, b7, w8, b8):
    def body(a1, m1, a2, m2, a3, m3, a4, m4, w5_ref, b5_ref,
             w6_ref, b6_ref, w7_ref, b7_ref, w8_ref, b8_ref,
             out_ref, pooled_v):
        bidx = pl.program_id(0)
        xcat = jnp.concatenate(
            [_lrelu(a1[0] + m1[0]), _lrelu(a2[0] + m2[0]),
             _lrelu(a3[0] + m3[0]), _lrelu(a4[0] + m4[0])], axis=0)
        h = jnp.dot(w5_ref[...].astype(jnp.bfloat16),
                    xcat.astype(jnp.bfloat16),
                    preferred_element_type=jnp.float32)
        h = _lrelu(h + b5_ref[...])
        mx = jnp.max(h, axis=1)
        av = jnp.sum(h, axis=1) * jnp.float32(1.0 / P)
        pooled_v[pl.ds(bidx, 1), :] = jnp.concatenate([mx, av], axis=0)[None, :]

        @pl.when(bidx == B - 1)
        def _():
            dn = (((1,), (1,)), ((), ()))
            z = lax.dot_general(pooled_v[...], w6_ref[...], dn,
                                preferred_element_type=jnp.float32)
            z = _lrelu(z + b6_ref[...])
            z = lax.dot_general(z, w7_ref[...], dn,
                                preferred_element_type=jnp.float32)
            z = _lrelu(z + b7_ref[...])
            z = lax.dot_general(z, w8_ref[...], dn,
                                preferred_element_type=jnp.float32)
            out_ref[...] = z + b8_ref[...]

    specs = []
    for c in (64, 64, 128, 256):
        specs.append(pl.BlockSpec((1, c, P), lambda b: (b, 0, 0)))
        specs.append(pl.BlockSpec((1, c, P), lambda b: (b, 0, 0)))
    for shp in ((1024, 512), (1024, 1), (512, 2048), (1, 512),
                (256, 512), (1, 256), (40, 256), (1, 40)):
        specs.append(pl.BlockSpec(shp, lambda b: (0, 0)))
    return pl.pallas_call(
        body,
        grid=(B,),
        in_specs=specs,
        out_specs=pl.BlockSpec((B, 40), lambda b: (0, 0)),
        out_shape=jax.ShapeDtypeStruct((B, 40), jnp.float32),
        scratch_shapes=[pltpu.VMEM((B, 2048), jnp.float32)],
    )(*ams, w5, b5, w6, b6, w7, b7, w8, b8)


def kernel(pos, batch, W1, g1, b1, W2, g2, b2, W3, g3, b3, W4, g4, b4,
           W5, g5, b5, W6, g6, b6, W7, bias7, g7, b7, W8, bias8):
    del batch
    pos_r = pos.reshape(B, P, 3)
    pos_c = jnp.transpose(pos_r, (0, 2, 1))

    idx_t = _knn(pos_r, pos_c)

    inv = 1.0 / jnp.sqrt(jnp.float32(1.0 + EPS))

    def prep(W, g, bb, cin):
        ws = W * (g * inv)[:, None]
        wi, wj = ws[:, :cin], ws[:, cin:]
        return wi - wj, wj, bb[:, None]

    ams = []
    xa, xm = pos_c, pos_c
    fuse = False
    for (W, g, bb, cin, cout) in ((W1, g1, b1, 3, 64),
                                  (W2, g2, b2, 64, 64),
                                  (W3, g3, b3, 64, 128),
                                  (W4, g4, b4, 128, 256)):
        wd, wj, bt = prep(W, g, bb, cin)
        a_t, bm_t = _layer_mm(xa, xm, wd, wj, bt, cin, cout, fuse)
        m_t = _sc_gather_max(bm_t, idx_t, cout)
        ams.extend([a_t, m_t])
        xa, xm, fuse = a_t, m_t, True

    w5s = W5 * (g5 * inv)[:, None]
    w6s = W6 * (g6 * inv)[:, None]
    w7s = W7 * (g7 * inv)[:, None]
    b7s = (bias7 * g7 * inv + b7)
    return _conv5_head(ams, w5s, b5[:, None], w6s, b6[None, :],
                       w7s, b7s[None, :], W8, bias8[None, :])


# speedup vs baseline: 1.8416x; 1.0067x over previous
import functools

import jax
import jax.numpy as jnp
from jax import lax
from jax.experimental import pallas as pl
from jax.experimental.pallas import tpu as pltpu
from jax.experimental.pallas import tpu_sc as plsc

K = 16
P = 1024
B = 8
EPS = 1e-5

NUM_SC_CORES = 2
NUM_SUBCORES = 16
NUM_TILES = NUM_SC_CORES * NUM_SUBCORES
LANES = 16


def _lrelu(x):
    return jnp.where(x >= 0, x, 0.2 * x)


_ROWS = 512


def _knn_body(pr_ref, pc_ref, out_ref):
    rows = pr_ref[0]
    cols = pc_ref[0]
    d = ((rows[:, 0:1] - cols[0:1, :]) ** 2
         + (rows[:, 1:2] - cols[1:2, :]) ** 2
         + (rows[:, 2:3] - cols[2:3, :]) ** 2)
    d = jnp.maximum(d, jnp.float32(2.0**-126))
    col_iota = lax.broadcasted_iota(jnp.int32, (_ROWS, P), 1)
    keys = lax.bitcast_convert_type(
        (lax.bitcast_convert_type(d, jnp.int32) & jnp.int32(~1023)) | col_iota,
        jnp.float32)
    inf = jnp.float32(jnp.inf)
    for k in range(K):
        m = jnp.min(keys, axis=1)
        out_ref[0, k, :] = (lax.bitcast_convert_type(m, jnp.int32)
                            & jnp.int32(1023))
        keys = jnp.where(keys == m[:, None], inf, keys)


def _knn(pos_r, pos_c):
    return pl.pallas_call(
        _knn_body,
        grid=(B, P // _ROWS),
        in_specs=[
            pl.BlockSpec((1, _ROWS, 3), lambda b, r: (b, r, 0)),
            pl.BlockSpec((1, 3, P), lambda b, r: (b, 0, 0)),
        ],
        out_specs=pl.BlockSpec((1, K, _ROWS), lambda b, r: (b, 0, r)),
        out_shape=jax.ShapeDtypeStruct((B, K, P), jnp.int32),
    )(pos_r, pos_c)


def _layer_mm(xa, xm, wd, wj, beta, cin, cout, fuse_in):
    def body(xa_ref, xm_ref, wd_ref, wj_ref, bt_ref, a_ref, bm_ref):
        if fuse_in:
            x = _lrelu(xa_ref[0] + xm_ref[0])
        else:
            x = xa_ref[0]
        if cin <= 4:
            a = jnp.zeros((cout, P), jnp.float32)
            bm = jnp.zeros((cout, P), jnp.float32)
            for c in range(cin):
                a = a + wd_ref[:, c:c + 1] * x[c:c + 1, :]
                bm = bm + wj_ref[:, c:c + 1] * x[c:c + 1, :]
        else:
            a = jnp.dot(wd_ref[...], x, preferred_element_type=jnp.float32)
            bm = jnp.dot(wj_ref[...], x, preferred_element_type=jnp.float32)
        a_ref[0] = a + bt_ref[...]
        bm_ref[0] = bm

    return pl.pallas_call(
        body,
        grid=(B,),
        in_specs=[
            pl.BlockSpec((1, cin, P), lambda b: (b, 0, 0)),
            pl.BlockSpec((1, cin, P), lambda b: (b, 0, 0)),
            pl.BlockSpec((cout, cin), lambda b: (0, 0)),
            pl.BlockSpec((cout, cin), lambda b: (0, 0)),
            pl.BlockSpec((cout, 1), lambda b: (0, 0)),
        ],
        out_specs=[
            pl.BlockSpec((1, cout, P), lambda b: (b, 0, 0)),
            pl.BlockSpec((1, cout, P), lambda b: (b, 0, 0)),
        ],
        out_shape=[
            jax.ShapeDtypeStruct((B, cout, P), jnp.float32),
            jax.ShapeDtypeStruct((B, cout, P), jnp.float32),
        ],
    )(xa, xm, wd, wj, beta)


_SC_CH = {64: 16, 128: 32, 256: 32}


def _sc_gather_max(bm_t, idx_t, cout):
    ch_sz = _SC_CH[cout]
    nch = cout // ch_sz
    ntasks = B * nch
    tasks_per_tile = ntasks // NUM_TILES
    mesh = plsc.VectorSubcoreMesh(core_axis_name="c", subcore_axis_name="s")

    @functools.partial(
        pl.kernel,
        mesh=mesh,
        out_type=jax.ShapeDtypeStruct((B, cout, P), jnp.float32),
        scratch_types=[
            pltpu.VMEM((ch_sz, P), jnp.float32),
            pltpu.VMEM((K, P), jnp.int32),
            pltpu.VMEM((ch_sz, P), jnp.float32),
        ],
        compiler_params=pltpu.CompilerParams(needs_layout_passes=False),
    )
    def k(bm_hbm, idx_hbm, out_hbm, bm_v, idx_v, out_v):
        wid = lax.axis_index("s") * NUM_SC_CORES + lax.axis_index("c")
        lane = lax.iota(jnp.int32, LANES)
        b = (wid * tasks_per_tile) // nch
        pltpu.sync_copy(idx_hbm.at[b], idx_v)
        for i in range(tasks_per_tile):
            t = wid * tasks_per_tile + i
            c0 = (t % nch) * ch_sz
            pltpu.sync_copy(bm_hbm.at[b, pl.ds(c0, ch_sz)], bm_v)

            def pg_body(pg, _):
                pvec = pg * LANES + lane
                jv = [plsc.load_gather(
                          idx_v, [jnp.full((LANES,), kk, jnp.int32), pvec])
                      for kk in range(K)]

                def c_body(c, _):
                    cvec = jnp.broadcast_to(c, (LANES,)).astype(jnp.int32)
                    m = jnp.full((LANES,), -jnp.inf, jnp.float32)
                    for kk in range(K):
                        v = plsc.load_gather(bm_v, [cvec, jv[kk]])
                        m = jnp.maximum(m, v)
                    plsc.store_scatter(out_v, [cvec, pvec], m)
                    return 0

                lax.fori_loop(0, ch_sz, c_body, 0)
                return 0

            lax.fori_loop(0, P // LANES, pg_body, 0)
            pltpu.sync_copy(out_v, out_hbm.at[b, pl.ds(c0, ch_sz)])

    return k(bm_t, idx_t)


def _conv5_part1(a1, m1, a2, m2, a3, m3, w5a):
    def body(a1_r, m1_r, a2_r, m2_r, a3_r, m3_r, w_ref, out_ref):
        xcat = jnp.concatenate(
            [_lrelu(a1_r[0] + m1_r[0]), _lrelu(a2_r[0] + m2_r[0]),
             _lrelu(a3_r[0] + m3_r[0])], axis=0)
        h = jnp.dot(w_ref[...].astype(jnp.bfloat16),
                    xcat.astype(jnp.bfloat16),
                    preferred_element_type=jnp.float32)
        out_ref[0] = h.astype(jnp.bfloat16)

    return pl.pallas_call(
        body,
        grid=(B,),
        in_specs=[
            pl.BlockSpec((1, 64, P), lambda b: (b, 0, 0)),
            pl.BlockSpec((1, 64, P), lambda b: (b, 0, 0)),
            pl.BlockSpec((1, 64, P), lambda b: (b, 0, 0)),
            pl.BlockSpec((1, 64, P), lambda b: (b, 0, 0)),
            pl.BlockSpec((1, 128, P), lambda b: (b, 0, 0)),
            pl.BlockSpec((1, 128, P), lambda b: (b, 0, 0)),
            pl.BlockSpec((1024, 256), lambda b: (0, 0)),
        ],
        out_specs=pl.BlockSpec((1, 1024, P), lambda b: (b, 0, 0)),
        out_shape=jax.ShapeDtypeStruct((B, 1024, P), jnp.bfloat16),
    )(a1, m1, a2, m2, a3, m3, w5a)


def _conv5_part2_head(h123, a4, m4, w5b, b5, w6, b6, w7, b7, w8, b8):
    def body(h123_r, a4_r, m4_r, w5_ref, b5_ref,
             w6_ref, b6_ref, w7_ref, b7_ref, w8_ref, b8_ref,
             out_ref, pooled_v):
        bidx = pl.program_id(0)
        x4 = _lrelu(a4_r[0] + m4_r[0])
        h = jnp.dot(w5_ref[...].astype(jnp.bfloat16),
                    x4.astype(jnp.bfloat16),
                    preferred_element_type=jnp.float32)
        h = _lrelu(h + h123_r[0].astype(jnp.float32) + b5_ref[...])
        mx = jnp.max(h, axis=1)
        av = jnp.sum(h, axis=1) * jnp.float32(1.0 / P)
        pooled_v[pl.ds(bidx, 1), :] = jnp.concatenate([mx, av], axis=0)[None, :]

        @pl.when(bidx == B - 1)
        def _():
            dn = (((1,), (1,)), ((), ()))
            z = lax.dot_general(pooled_v[...], w6_ref[...], dn,
                                preferred_element_type=jnp.float32)
            z = _lrelu(z + b6_ref[...])
            z = lax.dot_general(z, w7_ref[...], dn,
                                preferred_element_type=jnp.float32)
            z = _lrelu(z + b7_ref[...])
            z = lax.dot_general(z, w8_ref[...], dn,
                                preferred_element_type=jnp.float32)
            out_ref[...] = z + b8_ref[...]

    specs = [
        pl.BlockSpec((1, 1024, P), lambda b: (b, 0, 0)),
        pl.BlockSpec((1, 256, P), lambda b: (b, 0, 0)),
        pl.BlockSpec((1, 256, P), lambda b: (b, 0, 0)),
    ]
    for shp in ((1024, 256), (1024, 1), (512, 2048), (1, 512),
                (256, 512), (1, 256), (40, 256), (1, 40)):
        specs.append(pl.BlockSpec(shp, lambda b: (0, 0)))
    return pl.pallas_call(
        body,
        grid=(B,),
        in_specs=specs,
        out_specs=pl.BlockSpec((B, 40), lambda b: (0, 0)),
        out_shape=jax.ShapeDtypeStruct((B, 40), jnp.float32),
        scratch_shapes=[pltpu.VMEM((B, 2048), jnp.float32)],
    )(h123, a4, m4, w5b, b5, w6, b6, w7, b7, w8, b8)


def kernel(pos, batch, W1, g1, b1, W2, g2, b2, W3, g3, b3, W4, g4, b4,
           W5, g5, b5, W6, g6, b6, W7, bias7, g7, b7, W8, bias8):
    del batch
    pos_r = pos.reshape(B, P, 3)
    pos_c = jnp.transpose(pos_r, (0, 2, 1))

    idx_t = _knn(pos_r, pos_c)

    inv = 1.0 / jnp.sqrt(jnp.float32(1.0 + EPS))

    def prep(W, g, bb, cin):
        ws = W * (g * inv)[:, None]
        wi, wj = ws[:, :cin], ws[:, cin:]
        return wi - wj, wj, bb[:, None]

    ams = []
    xa, xm = pos_c, pos_c
    fuse = False
    for (W, g, bb, cin, cout) in ((W1, g1, b1, 3, 64),
                                  (W2, g2, b2, 64, 64),
                                  (W3, g3, b3, 64, 128),
                                  (W4, g4, b4, 128, 256)):
        wd, wj, bt = prep(W, g, bb, cin)
        a_t, bm_t = _layer_mm(xa, xm, wd, wj, bt, cin, cout, fuse)
        m_t = _sc_gather_max(bm_t, idx_t, cout)
        ams.extend([a_t, m_t])
        xa, xm, fuse = a_t, m_t, True

    w5s = W5 * (g5 * inv)[:, None]
    w6s = W6 * (g6 * inv)[:, None]
    w7s = W7 * (g7 * inv)[:, None]
    b7s = (bias7 * g7 * inv + b7)
    a1, m1, a2, m2, a3, m3, a4, m4 = ams
    h123 = _conv5_part1(a1, m1, a2, m2, a3, m3, w5s[:, :256])
    return _conv5_part2_head(h123, a4, m4, w5s[:, 256:], b5[:, None],
                             w6s, b6[None, :], w7s, b7s[None, :],
                             W8, bias8[None, :])


# speedup vs baseline: 1.8796x; 1.0206x over previous
import functools

import jax
import jax.numpy as jnp
from jax import lax
from jax.experimental import pallas as pl
from jax.experimental.pallas import tpu as pltpu
from jax.experimental.pallas import tpu_sc as plsc

K = 16
P = 1024
B = 8
EPS = 1e-5

NUM_SC_CORES = 2
NUM_SUBCORES = 16
NUM_TILES = NUM_SC_CORES * NUM_SUBCORES
LANES = 16


def _lrelu(x):
    return jnp.where(x >= 0, x, 0.2 * x)


_ROWS = 1024


def _knn_body(pr_ref, pc_ref, out_ref):
    rows = pr_ref[0]
    cols = pc_ref[0]
    d = ((rows[:, 0:1] - cols[0:1, :]) ** 2
         + (rows[:, 1:2] - cols[1:2, :]) ** 2
         + (rows[:, 2:3] - cols[2:3, :]) ** 2)
    d = jnp.maximum(d, jnp.float32(2.0**-126))
    col_iota = lax.broadcasted_iota(jnp.int32, (_ROWS, P), 1)
    keys = lax.bitcast_convert_type(
        (lax.bitcast_convert_type(d, jnp.int32) & jnp.int32(~1023)) | col_iota,
        jnp.float32)
    inf = jnp.float32(jnp.inf)
    for k in range(K):
        m = jnp.min(keys, axis=1)
        out_ref[0, k, :] = (lax.bitcast_convert_type(m, jnp.int32)
                            & jnp.int32(1023))
        keys = jnp.where(keys == m[:, None], inf, keys)


def _knn(pos_r, pos_c):
    return pl.pallas_call(
        _knn_body,
        grid=(B, P // _ROWS),
        in_specs=[
            pl.BlockSpec((1, _ROWS, 3), lambda b, r: (b, r, 0)),
            pl.BlockSpec((1, 3, P), lambda b, r: (b, 0, 0)),
        ],
        out_specs=pl.BlockSpec((1, K, _ROWS), lambda b, r: (b, 0, r)),
        out_shape=jax.ShapeDtypeStruct((B, K, P), jnp.int32),
    )(pos_r, pos_c)


def _layer_mm(xa, xm, wd, wj, beta, cin, cout, fuse_in):
    def body(xa_ref, xm_ref, wd_ref, wj_ref, bt_ref, a_ref, bm_ref):
        if fuse_in:
            x = _lrelu(xa_ref[0] + xm_ref[0])
        else:
            x = xa_ref[0]
        if cin <= 4:
            a = jnp.zeros((cout, P), jnp.float32)
            bm = jnp.zeros((cout, P), jnp.float32)
            for c in range(cin):
                a = a + wd_ref[:, c:c + 1] * x[c:c + 1, :]
                bm = bm + wj_ref[:, c:c + 1] * x[c:c + 1, :]
        else:
            a = jnp.dot(wd_ref[...], x, preferred_element_type=jnp.float32)
            bm = jnp.dot(wj_ref[...], x, preferred_element_type=jnp.float32)
        a_ref[0] = a + bt_ref[...]
        bm_ref[0] = bm

    return pl.pallas_call(
        body,
        grid=(B,),
        in_specs=[
            pl.BlockSpec((1, cin, P), lambda b: (b, 0, 0)),
            pl.BlockSpec((1, cin, P), lambda b: (b, 0, 0)),
            pl.BlockSpec((cout, cin), lambda b: (0, 0)),
            pl.BlockSpec((cout, cin), lambda b: (0, 0)),
            pl.BlockSpec((cout, 1), lambda b: (0, 0)),
        ],
        out_specs=[
            pl.BlockSpec((1, cout, P), lambda b: (b, 0, 0)),
            pl.BlockSpec((1, cout, P), lambda b: (b, 0, 0)),
        ],
        out_shape=[
            jax.ShapeDtypeStruct((B, cout, P), jnp.float32),
            jax.ShapeDtypeStruct((B, cout, P), jnp.float32),
        ],
    )(xa, xm, wd, wj, beta)


_SC_CH = {64: 16, 128: 32, 256: 32}


def _sc_gather_max(bm_t, idx_t, cout):
    ch_sz = _SC_CH[cout]
    nch = cout // ch_sz
    ntasks = B * nch
    tasks_per_tile = ntasks // NUM_TILES
    mesh = plsc.VectorSubcoreMesh(core_axis_name="c", subcore_axis_name="s")

    @functools.partial(
        pl.kernel,
        mesh=mesh,
        out_type=jax.ShapeDtypeStruct((B, cout, P), jnp.float32),
        scratch_types=[
            pltpu.VMEM((ch_sz, P), jnp.float32),
            pltpu.VMEM((ch_sz, P), jnp.float32),
            pltpu.VMEM((K, P), jnp.int32),
            pltpu.VMEM((ch_sz, P), jnp.float32),
            pltpu.SemaphoreType.DMA,
            pltpu.SemaphoreType.DMA,
        ],
        compiler_params=pltpu.CompilerParams(needs_layout_passes=False),
    )
    def k(bm_hbm, idx_hbm, out_hbm, bm_v0, bm_v1, idx_v, out_v, sem0, sem1):
        wid = lax.axis_index("s") * NUM_SC_CORES + lax.axis_index("c")
        lane = lax.iota(jnp.int32, LANES)
        b = (wid * tasks_per_tile) // nch
        bufs = [(bm_v0, sem0), (bm_v1, sem1)]

        def task_c0(i):
            return ((wid * tasks_per_tile + i) % nch) * ch_sz

        cp = pltpu.async_copy(bm_hbm.at[b, pl.ds(task_c0(0), ch_sz)],
                              bm_v0, sem0)
        pltpu.sync_copy(idx_hbm.at[b], idx_v)
        for i in range(tasks_per_tile):
            bm_v = bufs[i % 2][0]
            c0 = task_c0(i)
            cp.wait()
            if i + 1 < tasks_per_tile:
                nbuf, nsem = bufs[(i + 1) % 2]
                cp = pltpu.async_copy(
                    bm_hbm.at[b, pl.ds(task_c0(i + 1), ch_sz)], nbuf, nsem)

            def pg_body(pg, _):
                pvec = pg * LANES + lane
                jv = [plsc.load_gather(
                          idx_v, [jnp.full((LANES,), kk, jnp.int32), pvec])
                      for kk in range(K)]

                def c_body(c, _):
                    cvec = jnp.broadcast_to(c, (LANES,)).astype(jnp.int32)
                    m = jnp.full((LANES,), -jnp.inf, jnp.float32)
                    for kk in range(K):
                        v = plsc.load_gather(bm_v, [cvec, jv[kk]])
                        m = jnp.maximum(m, v)
                    plsc.store_scatter(out_v, [cvec, pvec], m)
                    return 0

                lax.fori_loop(0, ch_sz, c_body, 0)
                return 0

            lax.fori_loop(0, P // LANES, pg_body, 0)
            pltpu.sync_copy(out_v, out_hbm.at[b, pl.ds(c0, ch_sz)])

    return k(bm_t, idx_t)


def _conv5_part1(a1, m1, a2, m2, a3, m3, w5a):
    def body(a1_r, m1_r, a2_r, m2_r, a3_r, m3_r, w_ref, out_ref):
        xcat = jnp.concatenate(
            [_lrelu(a1_r[0] + m1_r[0]), _lrelu(a2_r[0] + m2_r[0]),
             _lrelu(a3_r[0] + m3_r[0])], axis=0)
        h = jnp.dot(w_ref[...].astype(jnp.bfloat16),
                    xcat.astype(jnp.bfloat16),
                    preferred_element_type=jnp.float32)
        out_ref[0] = h.astype(jnp.bfloat16)

    return pl.pallas_call(
        body,
        grid=(B,),
        in_specs=[
            pl.BlockSpec((1, 64, P), lambda b: (b, 0, 0)),
            pl.BlockSpec((1, 64, P), lambda b: (b, 0, 0)),
            pl.BlockSpec((1, 64, P), lambda b: (b, 0, 0)),
            pl.BlockSpec((1, 64, P), lambda b: (b, 0, 0)),
            pl.BlockSpec((1, 128, P), lambda b: (b, 0, 0)),
            pl.BlockSpec((1, 128, P), lambda b: (b, 0, 0)),
            pl.BlockSpec((1024, 256), lambda b: (0, 0)),
        ],
        out_specs=pl.BlockSpec((1, 1024, P), lambda b: (b, 0, 0)),
        out_shape=jax.ShapeDtypeStruct((B, 1024, P), jnp.bfloat16),
    )(a1, m1, a2, m2, a3, m3, w5a)


def _conv5_part2_head(h123, a4, m4, w5b, b5, w6, b6, w7, b7, w8, b8):
    def body(h123_r, a4_r, m4_r, w5_ref, b5_ref,
             w6_ref, b6_ref, w7_ref, b7_ref, w8_ref, b8_ref,
             out_ref, pooled_v):
        bidx = pl.program_id(0)
        x4 = _lrelu(a4_r[0] + m4_r[0])
        h = jnp.dot(w5_ref[...].astype(jnp.bfloat16),
                    x4.astype(jnp.bfloat16),
                    preferred_element_type=jnp.float32)
        h = _lrelu(h + h123_r[0].astype(jnp.float32) + b5_ref[...])
        mx = jnp.max(h, axis=1)
        av = jnp.sum(h, axis=1) * jnp.float32(1.0 / P)
        pooled_v[pl.ds(bidx, 1), :] = jnp.concatenate([mx, av], axis=0)[None, :]

        @pl.when(bidx == B - 1)
        def _():
            dn = (((1,), (1,)), ((), ()))
            z = lax.dot_general(pooled_v[...], w6_ref[...], dn,
                                preferred_element_type=jnp.float32)
            z = _lrelu(z + b6_ref[...])
            z = lax.dot_general(z, w7_ref[...], dn,
                                preferred_element_type=jnp.float32)
            z = _lrelu(z + b7_ref[...])
            z = lax.dot_general(z, w8_ref[...], dn,
                                preferred_element_type=jnp.float32)
            out_ref[...] = z + b8_ref[...]

    specs = [
        pl.BlockSpec((1, 1024, P), lambda b: (b, 0, 0)),
        pl.BlockSpec((1, 256, P), lambda b: (b, 0, 0)),
        pl.BlockSpec((1, 256, P), lambda b: (b, 0, 0)),
    ]
    for shp in ((1024, 256), (1024, 1), (512, 2048), (1, 512),
                (256, 512), (1, 256), (40, 256), (1, 40)):
        specs.append(pl.BlockSpec(shp, lambda b: (0, 0)))
    return pl.pallas_call(
        body,
        grid=(B,),
        in_specs=specs,
        out_specs=pl.BlockSpec((B, 40), lambda b: (0, 0)),
        out_shape=jax.ShapeDtypeStruct((B, 40), jnp.float32),
        scratch_shapes=[pltpu.VMEM((B, 2048), jnp.float32)],
    )(h123, a4, m4, w5b, b5, w6, b6, w7, b7, w8, b8)


def kernel(pos, batch, W1, g1, b1, W2, g2, b2, W3, g3, b3, W4, g4, b4,
           W5, g5, b5, W6, g6, b6, W7, bias7, g7, b7, W8, bias8):
    del batch
    pos_r = pos.reshape(B, P, 3)
    pos_c = jnp.transpose(pos_r, (0, 2, 1))

    idx_t = _knn(pos_r, pos_c)

    inv = 1.0 / jnp.sqrt(jnp.float32(1.0 + EPS))

    def prep(W, g, bb, cin):
        ws = W * (g * inv)[:, None]
        wi, wj = ws[:, :cin], ws[:, cin:]
        return wi - wj, wj, bb[:, None]

    ams = []
    xa, xm = pos_c, pos_c
    fuse = False
    for (W, g, bb, cin, cout) in ((W1, g1, b1, 3, 64),
                                  (W2, g2, b2, 64, 64),
                                  (W3, g3, b3, 64, 128),
                                  (W4, g4, b4, 128, 256)):
        wd, wj, bt = prep(W, g, bb, cin)
        a_t, bm_t = _layer_mm(xa, xm, wd, wj, bt, cin, cout, fuse)
        m_t = _sc_gather_max(bm_t, idx_t, cout)
        ams.extend([a_t, m_t])
        xa, xm, fuse = a_t, m_t, True

    w5s = W5 * (g5 * inv)[:, None]
    w6s = W6 * (g6 * inv)[:, None]
    w7s = W7 * (g7 * inv)[:, None]
    b7s = (bias7 * g7 * inv + b7)
    a1, m1, a2, m2, a3, m3, a4, m4 = ams
    h123 = _conv5_part1(a1, m1, a2, m2, a3, m3, w5s[:, :256])
    return _conv5_part2_head(h123, a4, m4, w5s[:, 256:], b5[:, None],
                             w6s, b6[None, :], w7s, b7s[None, :],
                             W8, bias8[None, :])
